# baseline (speedup 1.0000x reference)
import jax
import jax.numpy as jnp
from jax import lax
from jax.experimental import pallas as pl
from jax.experimental.pallas import tpu as pltpu
from jax.experimental.pallas import tpu_sc as plsc

_NUM_FIELDS = 26
_FIELD_DIM = 100000
_BATCH = 16384
_NC = 2
_NS = 16
_L = 16
_FPC = _NUM_FIELDS // _NC
_RPT = _BATCH // _NS
_CHUNK = 2048
_NCHUNK = _BATCH // _CHUNK


def _body(xt_hbm, w2_hbm, bias_hbm, out_hbm,
          tv, xi0, xi1, vc0, vc1, spm, acc, tmp0, tmp1, bias_v,
          sem, sem2, sem3):
    c = lax.axis_index("c")
    s = lax.axis_index("s")
    f = s * _NC + c

    @pl.when(s < _FPC)
    def _gather_phase():
        xis = (xi0, xi1)
        vcs = (vc0, vc1)
        h1 = pltpu.async_copy(w2_hbm.at[f], tv, sem)
        hx = pltpu.async_copy(xt_hbm.at[f, pl.ds(0, _CHUNK)], xis[0], sem2)
        h1.wait()
        pubs = []
        for chunk in range(_NCHUNK):
            hx.wait()
            if chunk + 1 < _NCHUNK:
                hx = pltpu.async_copy(
                    xt_hbm.at[f, pl.ds((chunk + 1) * _CHUNK, _CHUNK)],
                    xis[(chunk + 1) % 2], sem2)
            xc = xis[chunk % 2]
            vc = vcs[chunk % 2]
            if chunk >= 2:
                pubs[chunk - 2].wait()
            for j in range(_CHUNK // _L):
                idx = xc[pl.ds(j * _L, _L)]
                vc[pl.ds(j * _L, _L)] = plsc.load_gather(tv, [idx])
            pubs.append(pltpu.async_copy(
                vc, spm.at[pl.ds(s * _BATCH + chunk * _CHUNK, _CHUNK)], sem3))
        for p in pubs[-2:]:
            p.wait()

    plsc.subcore_barrier()

    rbase = s * _RPT
    pltpu.sync_copy(bias_hbm, bias_v)
    bias_vec = bias_v[...] * (1 - c).astype(jnp.float32)
    for j in range(_RPT // _L):
        acc[pl.ds(j * _L, _L)] = bias_vec

    tmps = (tmp0, tmp1)
    h = pltpu.async_copy(spm.at[pl.ds(rbase, _RPT)], tmps[0], sem)
    for k in range(_FPC):
        h.wait()
        if k + 1 < _FPC:
            h = pltpu.async_copy(
                spm.at[pl.ds((k + 1) * _BATCH + rbase, _RPT)],
                tmps[(k + 1) % 2], sem)
        t = tmps[k % 2]
        for j in range(_RPT // _L):
            acc[pl.ds(j * _L, _L)] = acc[pl.ds(j * _L, _L)] + t[pl.ds(j * _L, _L)]

    pltpu.sync_copy(acc, out_hbm.at[c, pl.ds(rbase, _RPT)])


def kernel(x, W, bias):
    xt = x.T
    w2 = W.reshape(_NUM_FIELDS, _FIELD_DIM)
    bias16 = jnp.broadcast_to(bias, (_L,)).astype(jnp.float32)

    mesh = plsc.VectorSubcoreMesh(
        core_axis_name="c", subcore_axis_name="s",
        num_cores=_NC, num_subcores=_NS,
    )
    fn = pl.kernel(
        _body,
        out_type=jax.ShapeDtypeStruct((_NC, _BATCH), jnp.float32),
        mesh=mesh,
        compiler_params=pltpu.CompilerParams(needs_layout_passes=False),
        scratch_types=[
            pltpu.VMEM((_FIELD_DIM,), jnp.float32),
            pltpu.VMEM((_CHUNK,), jnp.int32),
            pltpu.VMEM((_CHUNK,), jnp.int32),
            pltpu.VMEM((_CHUNK,), jnp.float32),
            pltpu.VMEM((_CHUNK,), jnp.float32),
            pltpu.VMEM_SHARED((_FPC * _BATCH,), jnp.float32),
            pltpu.VMEM((_RPT,), jnp.float32),
            pltpu.VMEM((_RPT,), jnp.float32),
            pltpu.VMEM((_RPT,), jnp.float32),
            pltpu.VMEM((_L,), jnp.float32),
            pltpu.SemaphoreType.DMA,
            pltpu.SemaphoreType.DMA,
            pltpu.SemaphoreType.DMA,
        ],
    )
    partial = fn(xt, w2, bias16)
    return partial[0] + partial[1]


# speedup vs baseline: 1.0000x; 1.0000x over previous
import jax
import jax.numpy as jnp
from jax import lax
from jax.experimental import pallas as pl
from jax.experimental.pallas import tpu as pltpu
from jax.experimental.pallas import tpu_sc as plsc

_NUM_FIELDS = 26
_FIELD_DIM = 100000
_BATCH = 16384
_NC = 2
_NS = 16
_L = 16
_FPC = _NUM_FIELDS // _NC
_RPT = _BATCH // _NS
_CHUNK = 2048
_NCHUNK = _BATCH // _CHUNK


def _body(xt_hbm, w2_hbm, bias_hbm, out_hbm,
          tv, xi0, xi1, vc0, vc1, spm, acc, tmp0, tmp1, bias_v,
          sem, sem2, sem3):
    c = lax.axis_index("c")
    s = lax.axis_index("s")
    f = s * _NC + c

    @pl.when(s < _FPC)
    def _gather_phase():
        xis = (xi0, xi1)
        vcs = (vc0, vc1)
        with jax.named_scope("phase_stage"):
            h1 = pltpu.async_copy(w2_hbm.at[f], tv, sem)
            hx = pltpu.async_copy(xt_hbm.at[f, pl.ds(0, _CHUNK)], xis[0], sem2)
            h1.wait()
        with jax.named_scope("phase_gather"):
            pubs = []
            for chunk in range(_NCHUNK):
                hx.wait()
                if chunk + 1 < _NCHUNK:
                    hx = pltpu.async_copy(
                        xt_hbm.at[f, pl.ds((chunk + 1) * _CHUNK, _CHUNK)],
                        xis[(chunk + 1) % 2], sem2)
                xc = xis[chunk % 2]
                vc = vcs[chunk % 2]
                if chunk >= 2:
                    pubs[chunk - 2].wait()
                for j in range(_CHUNK // _L):
                    idx = xc[pl.ds(j * _L, _L)]
                    vc[pl.ds(j * _L, _L)] = plsc.load_gather(tv, [idx])
                pubs.append(pltpu.async_copy(
                    vc, spm.at[pl.ds(s * _BATCH + chunk * _CHUNK, _CHUNK)], sem3))
            for p in pubs[-2:]:
                p.wait()

    with jax.named_scope("phase_barrier"):
        plsc.subcore_barrier()

    rbase = s * _RPT
    pltpu.sync_copy(bias_hbm, bias_v)
    bias_vec = bias_v[...] * (1 - c).astype(jnp.float32)
    for j in range(_RPT // _L):
        acc[pl.ds(j * _L, _L)] = bias_vec

    with jax.named_scope("phase_reduce"):
        tmps = (tmp0, tmp1)
        h = pltpu.async_copy(spm.at[pl.ds(rbase, _RPT)], tmps[0], sem)
        for k in range(_FPC):
            h.wait()
            if k + 1 < _FPC:
                h = pltpu.async_copy(
                    spm.at[pl.ds((k + 1) * _BATCH + rbase, _RPT)],
                    tmps[(k + 1) % 2], sem)
            t = tmps[k % 2]
            for j in range(_RPT // _L):
                acc[pl.ds(j * _L, _L)] = acc[pl.ds(j * _L, _L)] + t[pl.ds(j * _L, _L)]

    with jax.named_scope("phase_out"):
        pltpu.sync_copy(acc, out_hbm.at[c, pl.ds(rbase, _RPT)])


def kernel(x, W, bias):
    xt = x.T
    w2 = W.reshape(_NUM_FIELDS, _FIELD_DIM)
    bias16 = jnp.broadcast_to(bias, (_L,)).astype(jnp.float32)

    mesh = plsc.VectorSubcoreMesh(
        core_axis_name="c", subcore_axis_name="s",
        num_cores=_NC, num_subcores=_NS,
    )
    fn = pl.kernel(
        _body,
        out_type=jax.ShapeDtypeStruct((_NC, _BATCH), jnp.float32),
        mesh=mesh,
        compiler_params=pltpu.CompilerParams(needs_layout_passes=False),
        scratch_types=[
            pltpu.VMEM((_FIELD_DIM,), jnp.float32),
            pltpu.VMEM((_CHUNK,), jnp.int32),
            pltpu.VMEM((_CHUNK,), jnp.int32),
            pltpu.VMEM((_CHUNK,), jnp.float32),
            pltpu.VMEM((_CHUNK,), jnp.float32),
            pltpu.VMEM_SHARED((_FPC * _BATCH,), jnp.float32),
            pltpu.VMEM((_RPT,), jnp.float32),
            pltpu.VMEM((_RPT,), jnp.float32),
            pltpu.VMEM((_RPT,), jnp.float32),
            pltpu.VMEM((_L,), jnp.float32),
            pltpu.SemaphoreType.DMA,
            pltpu.SemaphoreType.DMA,
            pltpu.SemaphoreType.DMA,
        ],
    )
    partial = fn(xt, w2, bias16)
    return partial[0] + partial[1]


# speedup vs baseline: 1.0443x; 1.0443x over previous
import jax
import jax.numpy as jnp
from jax import lax
from jax.experimental import pallas as pl
from jax.experimental.pallas import tpu as pltpu
from jax.experimental.pallas import tpu_sc as plsc

_NUM_FIELDS = 26
_FIELD_DIM = 100000
_BATCH = 16384
_NC = 2
_NS = 16
_L = 16
_FPC = _NUM_FIELDS // _NC
_RPT = _BATCH // _NS
_CHUNK = 2048
_NCHUNK = _BATCH // _CHUNK


def _body(xt_hbm, w2_hbm, bias_hbm, out_hbm,
          tv, xi0, xi1, vc0, vc1, spm, acc, tmp0, tmp1, bias_v,
          sem, sem2, sem3, semb):
    c = lax.axis_index("c")
    s = lax.axis_index("s")
    f = s * _NC + c

    hb = pltpu.async_copy(bias_hbm, bias_v, semb)

    @pl.when(s < _FPC)
    def _gather_phase():
        xis = (xi0, xi1)
        vcs = (vc0, vc1)
        with jax.named_scope("phase_stage"):
            h1 = pltpu.async_copy(w2_hbm.at[f], tv, sem)
            hx = pltpu.async_copy(xt_hbm.at[f, pl.ds(0, _CHUNK)], xis[0], sem2)
            h1.wait()
        with jax.named_scope("phase_gather"):
            pubs = []
            for chunk in range(_NCHUNK):
                hx.wait()
                if chunk + 1 < _NCHUNK:
                    hx = pltpu.async_copy(
                        xt_hbm.at[f, pl.ds((chunk + 1) * _CHUNK, _CHUNK)],
                        xis[(chunk + 1) % 2], sem2)
                xc = xis[chunk % 2]
                vc = vcs[chunk % 2]
                if chunk >= 2:
                    pubs[chunk - 2].wait()

                @plsc.parallel_loop(0, _CHUNK, step=_L, unroll=8)
                def _gather(i):
                    vc[pl.ds(i, _L)] = plsc.load_gather(tv, [xc[pl.ds(i, _L)]])

                pubs.append(pltpu.async_copy(
                    vc, spm.at[pl.ds(s * _BATCH + chunk * _CHUNK, _CHUNK)],
                    sem3))
            for p in pubs[-2:]:
                p.wait()

    with jax.named_scope("phase_barrier"):
        plsc.subcore_barrier()

    rbase = s * _RPT
    with jax.named_scope("phase_reduce"):
        hb.wait()
        bias_vec = bias_v[...] * (1 - c).astype(jnp.float32)

        @plsc.parallel_loop(0, _RPT, step=_L, unroll=8)
        def _init(j):
            acc[pl.ds(j, _L)] = bias_vec

        tmps = (tmp0, tmp1)
        h = pltpu.async_copy(spm.at[pl.ds(rbase, _RPT)], tmps[0], sem)
        for k in range(_FPC):
            h.wait()
            if k + 1 < _FPC:
                h = pltpu.async_copy(
                    spm.at[pl.ds((k + 1) * _BATCH + rbase, _RPT)],
                    tmps[(k + 1) % 2], sem)
            t = tmps[k % 2]

            @plsc.parallel_loop(0, _RPT, step=_L, unroll=8)
            def _red(j):
                acc[pl.ds(j, _L)] = acc[pl.ds(j, _L)] + t[pl.ds(j, _L)]

    with jax.named_scope("phase_out"):
        pltpu.sync_copy(acc, out_hbm.at[c, pl.ds(rbase, _RPT)])


def kernel(x, W, bias):
    xt = x.T
    w2 = W.reshape(_NUM_FIELDS, _FIELD_DIM)
    bias16 = jnp.broadcast_to(bias, (_L,)).astype(jnp.float32)

    mesh = plsc.VectorSubcoreMesh(
        core_axis_name="c", subcore_axis_name="s",
        num_cores=_NC, num_subcores=_NS,
    )
    fn = pl.kernel(
        _body,
        out_type=jax.ShapeDtypeStruct((_NC, _BATCH), jnp.float32),
        mesh=mesh,
        compiler_params=pltpu.CompilerParams(needs_layout_passes=False),
        scratch_types=[
            pltpu.VMEM((_FIELD_DIM,), jnp.float32),
            pltpu.VMEM((_CHUNK,), jnp.int32),
            pltpu.VMEM((_CHUNK,), jnp.int32),
            pltpu.VMEM((_CHUNK,), jnp.float32),
            pltpu.VMEM((_CHUNK,), jnp.float32),
            pltpu.VMEM_SHARED((_FPC * _BATCH,), jnp.float32),
            pltpu.VMEM((_RPT,), jnp.float32),
            pltpu.VMEM((_RPT,), jnp.float32),
            pltpu.VMEM((_RPT,), jnp.float32),
            pltpu.VMEM((_L,), jnp.float32),
            pltpu.SemaphoreType.DMA,
            pltpu.SemaphoreType.DMA,
            pltpu.SemaphoreType.DMA,
            pltpu.SemaphoreType.DMA,
        ],
    )
    partial = fn(xt, w2, bias16)
    return partial[0] + partial[1]


# speedup vs baseline: 1.0486x; 1.0041x over previous
import jax
import jax.numpy as jnp
from jax import lax
from jax.experimental import pallas as pl
from jax.experimental.pallas import tpu as pltpu
from jax.experimental.pallas import tpu_sc as plsc

_NUM_FIELDS = 26
_FIELD_DIM = 100000
_BATCH = 16384
_NC = 2
_NS = 16
_L = 16
_FPC = _NUM_FIELDS // _NC
_RPT = _BATCH // _NS
_CHUNK = 2048
_NCHUNK = _BATCH // _CHUNK


def _body(xt_hbm, w2_hbm, bias_hbm, out_hbm,
          tv, xi0, xi1, vc0, vc1, vc2, vc3, spm, acc,
          tmp0, tmp1, tmp2, tmp3, bias_v,
          sem, sem2, sem3, semb):
    c = lax.axis_index("c")
    s = lax.axis_index("s")
    f = s * _NC + c

    hb = pltpu.async_copy(bias_hbm, bias_v, semb)

    @pl.when(s < _FPC)
    def _gather_phase():
        xis = (xi0, xi1)
        vcs = (vc0, vc1, vc2, vc3)
        with jax.named_scope("phase_stage"):
            h1 = pltpu.async_copy(w2_hbm.at[f], tv, sem)
            hx = pltpu.async_copy(xt_hbm.at[f, pl.ds(0, _CHUNK)], xis[0], sem2)
            h1.wait()
        with jax.named_scope("phase_gather"):
            pubs = []
            for chunk in range(_NCHUNK):
                hx.wait()
                if chunk + 1 < _NCHUNK:
                    hx = pltpu.async_copy(
                        xt_hbm.at[f, pl.ds((chunk + 1) * _CHUNK, _CHUNK)],
                        xis[(chunk + 1) % 2], sem2)
                xc = xis[chunk % 2]
                vc = vcs[chunk % 4]
                if chunk >= 4:
                    pubs[chunk - 4].wait()

                @plsc.parallel_loop(0, _CHUNK, step=_L, unroll=8)
                def _gather(i):
                    vc[pl.ds(i, _L)] = plsc.load_gather(tv, [xc[pl.ds(i, _L)]])

                pubs.append(pltpu.async_copy(
                    vc, spm.at[pl.ds(s * _BATCH + chunk * _CHUNK, _CHUNK)],
                    sem3))
            for p in pubs[-4:]:
                p.wait()

    with jax.named_scope("phase_barrier"):
        plsc.subcore_barrier()

    rbase = s * _RPT
    with jax.named_scope("phase_reduce"):
        hb.wait()
        bias_vec = bias_v[...] * (1 - c).astype(jnp.float32)

        @plsc.parallel_loop(0, _RPT, step=_L, unroll=8)
        def _init(j):
            acc[pl.ds(j, _L)] = bias_vec

        tmps = (tmp0, tmp1, tmp2, tmp3)
        hs = [pltpu.async_copy(
                  spm.at[pl.ds(k * _BATCH + rbase, _RPT)], tmps[k], sem)
              for k in range(3)]
        for k in range(_FPC):
            hs[k].wait()
            if k + 3 < _FPC:
                hs.append(pltpu.async_copy(
                    spm.at[pl.ds((k + 3) * _BATCH + rbase, _RPT)],
                    tmps[(k + 3) % 4], sem))
            t = tmps[k % 4]

            @plsc.parallel_loop(0, _RPT, step=_L, unroll=8)
            def _red(j):
                acc[pl.ds(j, _L)] = acc[pl.ds(j, _L)] + t[pl.ds(j, _L)]

    with jax.named_scope("phase_out"):
        pltpu.sync_copy(acc, out_hbm.at[c, pl.ds(rbase, _RPT)])


def kernel(x, W, bias):
    xt = x.T
    w2 = W.reshape(_NUM_FIELDS, _FIELD_DIM)
    bias16 = jnp.broadcast_to(bias, (_L,)).astype(jnp.float32)

    mesh = plsc.VectorSubcoreMesh(
        core_axis_name="c", subcore_axis_name="s",
        num_cores=_NC, num_subcores=_NS,
    )
    fn = pl.kernel(
        _body,
        out_type=jax.ShapeDtypeStruct((_NC, _BATCH), jnp.float32),
        mesh=mesh,
        compiler_params=pltpu.CompilerParams(needs_layout_passes=False),
        scratch_types=[
            pltpu.VMEM((_FIELD_DIM,), jnp.float32),
            pltpu.VMEM((_CHUNK,), jnp.int32),
            pltpu.VMEM((_CHUNK,), jnp.int32),
            pltpu.VMEM((_CHUNK,), jnp.float32),
            pltpu.VMEM((_CHUNK,), jnp.float32),
            pltpu.VMEM((_CHUNK,), jnp.float32),
            pltpu.VMEM((_CHUNK,), jnp.float32),
            pltpu.VMEM_SHARED((_FPC * _BATCH,), jnp.float32),
            pltpu.VMEM((_RPT,), jnp.float32),
            pltpu.VMEM((_RPT,), jnp.float32),
            pltpu.VMEM((_RPT,), jnp.float32),
            pltpu.VMEM((_RPT,), jnp.float32),
            pltpu.VMEM((_RPT,), jnp.float32),
            pltpu.VMEM((_L,), jnp.float32),
            pltpu.SemaphoreType.DMA,
            pltpu.SemaphoreType.DMA,
            pltpu.SemaphoreType.DMA,
            pltpu.SemaphoreType.DMA,
        ],
    )
    partial = fn(xt, w2, bias16)
    return partial[0] + partial[1]


# speedup vs baseline: 1.0618x; 1.0126x over previous
import jax
import jax.numpy as jnp
from jax import lax
from jax.experimental import pallas as pl
from jax.experimental.pallas import tpu as pltpu
from jax.experimental.pallas import tpu_sc as plsc

_NUM_FIELDS = 26
_FIELD_DIM = 100000
_BATCH = 16384
_NC = 2
_NS = 16
_L = 16
_FPC = _NUM_FIELDS // _NC
_RPT = _BATCH // _NS
_CHUNK = 2048
_NCHUNK = _BATCH // _CHUNK


def _body(xt_hbm, w2_hbm, bias_hbm, out_hbm,
          tv, xi0, xi1, xi2, vc0, vc1, vc2, spm, acc,
          tmp0, tmp1, tmp2, tmp3, bias_v,
          sem, sem2, sem3, semb):
    c = lax.axis_index("c")
    s = lax.axis_index("s")
    f = s * _NC + c

    hb = pltpu.async_copy(bias_hbm, bias_v, semb)

    @pl.when(s < _FPC)
    def _gather_phase():
        xis = (xi0, xi1, xi2)
        vcs = (vc0, vc1, vc2)
        with jax.named_scope("phase_stage"):
            h1 = pltpu.async_copy(w2_hbm.at[f], tv, sem)
            hxs = [pltpu.async_copy(
                       xt_hbm.at[f, pl.ds(k * _CHUNK, _CHUNK)], xis[k], sem2)
                   for k in range(2)]
            h1.wait()
        with jax.named_scope("phase_gather"):
            pubs = []
            for chunk in range(_NCHUNK):
                hxs[chunk].wait()
                if chunk + 2 < _NCHUNK:
                    hxs.append(pltpu.async_copy(
                        xt_hbm.at[f, pl.ds((chunk + 2) * _CHUNK, _CHUNK)],
                        xis[(chunk + 2) % 3], sem2))
                xc = xis[chunk % 3]
                vc = vcs[chunk % 3]
                if chunk >= 3:
                    pubs[chunk - 3].wait()

                @plsc.parallel_loop(0, _CHUNK, step=_L, unroll=16)
                def _gather(i):
                    vc[pl.ds(i, _L)] = plsc.load_gather(tv, [xc[pl.ds(i, _L)]])

                pubs.append(pltpu.async_copy(
                    vc, spm.at[pl.ds(s * _BATCH + chunk * _CHUNK, _CHUNK)],
                    sem3))
            for p in pubs[-3:]:
                p.wait()

    with jax.named_scope("phase_barrier"):
        plsc.subcore_barrier()

    rbase = s * _RPT
    with jax.named_scope("phase_reduce"):
        hb.wait()
        bias_vec = bias_v[...] * (1 - c).astype(jnp.float32)

        @plsc.parallel_loop(0, _RPT, step=_L, unroll=8)
        def _init(j):
            acc[pl.ds(j, _L)] = bias_vec

        tmps = (tmp0, tmp1, tmp2, tmp3)
        hs = [pltpu.async_copy(
                  spm.at[pl.ds(k * _BATCH + rbase, _RPT)], tmps[k], sem)
              for k in range(3)]
        for k in range(_FPC):
            hs[k].wait()
            if k + 3 < _FPC:
                hs.append(pltpu.async_copy(
                    spm.at[pl.ds((k + 3) * _BATCH + rbase, _RPT)],
                    tmps[(k + 3) % 4], sem))
            t = tmps[k % 4]

            @plsc.parallel_loop(0, _RPT, step=_L, unroll=8)
            def _red(j):
                acc[pl.ds(j, _L)] = acc[pl.ds(j, _L)] + t[pl.ds(j, _L)]

    with jax.named_scope("phase_out"):
        pltpu.sync_copy(acc, out_hbm.at[c, pl.ds(rbase, _RPT)])


def kernel(x, W, bias):
    xt = x.T
    w2 = W.reshape(_NUM_FIELDS, _FIELD_DIM)
    bias16 = jnp.broadcast_to(bias, (_L,)).astype(jnp.float32)

    mesh = plsc.VectorSubcoreMesh(
        core_axis_name="c", subcore_axis_name="s",
        num_cores=_NC, num_subcores=_NS,
    )
    fn = pl.kernel(
        _body,
        out_type=jax.ShapeDtypeStruct((_NC, _BATCH), jnp.float32),
        mesh=mesh,
        compiler_params=pltpu.CompilerParams(needs_layout_passes=False),
        scratch_types=[
            pltpu.VMEM((_FIELD_DIM,), jnp.float32),
            pltpu.VMEM((_CHUNK,), jnp.int32),
            pltpu.VMEM((_CHUNK,), jnp.int32),
            pltpu.VMEM((_CHUNK,), jnp.int32),
            pltpu.VMEM((_CHUNK,), jnp.float32),
            pltpu.VMEM((_CHUNK,), jnp.float32),
            pltpu.VMEM((_CHUNK,), jnp.float32),
            pltpu.VMEM_SHARED((_FPC * _BATCH,), jnp.float32),
            pltpu.VMEM((_RPT,), jnp.float32),
            pltpu.VMEM((_RPT,), jnp.float32),
            pltpu.VMEM((_RPT,), jnp.float32),
            pltpu.VMEM((_RPT,), jnp.float32),
            pltpu.VMEM((_RPT,), jnp.float32),
            pltpu.VMEM((_L,), jnp.float32),
            pltpu.SemaphoreType.DMA,
            pltpu.SemaphoreType.DMA,
            pltpu.SemaphoreType.DMA,
            pltpu.SemaphoreType.DMA,
        ],
    )
    partial = fn(xt, w2, bias16)
    return partial[0] + partial[1]
